# Initial kernel scaffold; baseline (speedup 1.0000x reference)
import jax
import jax.numpy as jnp
from jax.experimental import pallas as pl


def kernel(predictions, targets):
    raise NotImplementedError("write your pallas kernel here")


# speedup vs baseline: 3.0307x; 3.0307x over previous
import functools
import math

import jax
import jax.numpy as jnp
from jax import lax
from jax.experimental import pallas as pl
from jax.experimental.pallas import tpu as pltpu
from jax.experimental.pallas import tpu_sc as plsc

_Bb = 2
_Cc = 80
_LC = 5.0
_LN = 0.1
_BS = 32
_S = 128
_NT = 100
_CELL = 1.0 / _S
_NCHUNK = 7
_NPAD = _NCHUNK * 16

_NC = 2
_NS = 16
_L = 16


def _sigmoid(x):
    return 1.0 / (1.0 + jnp.exp(-x))


def _sc_body(pred_rows, tgt, meta_out, iou_out, box_out,
             tgt_v, idx_v, rows_v, mbuf_v, meta_v, iou_v, box_v, sem):
    b = lax.axis_index("s") * _NC + lax.axis_index("c")
    lanes = lax.iota(jnp.int32, _L)
    zf = jnp.zeros((_L,), jnp.float32)
    zi = jnp.zeros((_L,), jnp.int32)

    def _zero16(i, _):
        iou_v[pl.ds(i * _L, _L)] = zf
        meta_v[pl.ds(i * _L, _L)] = zi
        return 0
    lax.fori_loop(0, (_S * _S) // _L, _zero16, 0)

    def _zero_box(i, _):
        box_v[pl.ds(i * _L, _L)] = zf
        return 0
    lax.fori_loop(0, (4 * _S * _S) // _L, _zero_box, 0)

    pltpu.sync_copy(tgt.at[b], tgt_v)

    def _cells(c):
        lane = lanes + c * _L
        lv = jnp.minimum(lane, _NT - 1)
        base = lv * 5
        cx = plsc.load_gather(tgt_v, [base + 1])
        cy = plsc.load_gather(tgt_v, [base + 2])
        col_f = cx * float(_S)
        row_f = cy * float(_S)
        cols_c = jnp.clip(col_f.astype(jnp.int32), 0, _S - 1)
        rows_c = jnp.clip(row_f.astype(jnp.int32), 0, _S - 1)
        valid = lane < _NT
        return lane, lv, col_f, row_f, cols_c, rows_c, valid

    def _p1(c, _):
        _, _, _, _, cols_c, rows_c, _ = _cells(c)
        cell_idx = rows_c * _S + cols_c
        idx_v[pl.ds(c * _L, _L)] = (b * (_S * _S) + cell_idx) * 9
        return 0
    lax.fori_loop(0, _NCHUNK, _p1, 0)

    pltpu.async_copy(pred_rows.at[idx_v], rows_v, sem).wait()

    def _p2(c, _):
        lane, lv, col_f, row_f, cols_c, rows_c, valid = _cells(c)
        base = lv * 5
        cls_i = plsc.load_gather(tgt_v, [base + 0]).astype(jnp.int32)
        w = plsc.load_gather(tgt_v, [base + 3])
        h = plsc.load_gather(tgt_v, [base + 4])
        cx = col_f * _CELL
        cy = row_f * _CELL
        cx_rel = col_f - cols_c.astype(jnp.float32)
        cy_rel = row_f - rows_c.astype(jnp.float32)

        ch = [plsc.load_gather(rows_v, [lane, jnp.full((_L,), k, jnp.int32)])
              for k in range(10)]

        gx1 = cx - w * 0.5
        gy1 = cy - h * 0.5
        gx2 = cx + w * 0.5
        gy2 = cy + h * 0.5
        garea = jnp.clip(gx2 - gx1, 0.0, None) * jnp.clip(gy2 - gy1, 0.0, None)

        def _iou_one(tx, ty, tw, th):
            p_cx = jnp.clip((_sigmoid(tx) + cols_c.astype(jnp.float32)) * _CELL, 0.0, 1.0)
            p_cy = jnp.clip((_sigmoid(ty) + rows_c.astype(jnp.float32)) * _CELL, 0.0, 1.0)
            p_w = jnp.clip(jnp.exp(jnp.clip(tw, -10.0, 10.0)) * _CELL, 1e-6, 1.0)
            p_h = jnp.clip(jnp.exp(jnp.clip(th, -10.0, 10.0)) * _CELL, 1e-6, 1.0)
            px1 = p_cx - p_w * 0.5
            py1 = p_cy - p_h * 0.5
            px2 = p_cx + p_w * 0.5
            py2 = p_cy + p_h * 0.5
            ix1 = jnp.maximum(px1, gx1)
            iy1 = jnp.maximum(py1, gy1)
            ix2 = jnp.minimum(px2, gx2)
            iy2 = jnp.minimum(py2, gy2)
            inter = jnp.clip(ix2 - ix1, 0.0, None) * jnp.clip(iy2 - iy1, 0.0, None)
            union = (jnp.clip(px2 - px1, 0.0, None) * jnp.clip(py2 - py1, 0.0, None)
                     + garea - inter)
            return inter / (union + 1e-6)

        iou0 = _iou_one(ch[0], ch[1], ch[2], ch[3])
        iou1 = _iou_one(ch[5], ch[6], ch[7], ch[8])
        best_js = (iou1 > iou0).astype(jnp.int32)
        best_iou = jnp.maximum(iou0, iou1)
        mval = 1 | (best_js << 1) | (cls_i << 2)
        mbuf_v[pl.ds(c * _L, _L)] = mval

        cell_idx = rows_c * _S + cols_c
        plsc.store_scatter(iou_v, [cell_idx], best_iou, mask=valid)
        plsc.store_scatter(box_v, [cell_idx], cx_rel, mask=valid)
        plsc.store_scatter(box_v, [cell_idx + _S * _S], cy_rel, mask=valid)
        plsc.store_scatter(box_v, [cell_idx + 2 * _S * _S], w, mask=valid)
        plsc.store_scatter(box_v, [cell_idx + 3 * _S * _S], h, mask=valid)
        plsc.store_scatter(meta_v, [cell_idx], mval, mask=valid)
        return 0
    lax.fori_loop(0, _NCHUNK, _p2, 0)

    def _p3(c, _):
        _, _, col_f, _, cols_c, rows_c, valid = _cells(c)
        frac = col_f - cols_c.astype(jnp.float32)
        dx = jnp.where(frac >= 0.5, 1, -1).astype(jnp.int32)
        nc = cols_c + dx
        ok = valid & (nc >= 0) & (nc < _S)
        mval = mbuf_v[pl.ds(c * _L, _L)]
        plsc.store_scatter(meta_v, [rows_c * _S + jnp.clip(nc, 0, _S - 1)], mval, mask=ok)
        return 0
    lax.fori_loop(0, _NCHUNK, _p3, 0)

    def _p4(c, _):
        _, _, _, row_f, cols_c, rows_c, valid = _cells(c)
        frac = row_f - rows_c.astype(jnp.float32)
        dy = jnp.where(frac >= 0.5, 1, -1).astype(jnp.int32)
        nr = rows_c + dy
        ok = valid & (nr >= 0) & (nr < _S)
        mval = mbuf_v[pl.ds(c * _L, _L)]
        plsc.store_scatter(meta_v, [jnp.clip(nr, 0, _S - 1) * _S + cols_c], mval, mask=ok)
        return 0
    lax.fori_loop(0, _NCHUNK, _p4, 0)

    pltpu.sync_copy(meta_v, meta_out.at[b])
    pltpu.sync_copy(iou_v, iou_out.at[b])
    pltpu.sync_copy(box_v, box_out.at[b])


def _sc_build(pred_rows, tgt_pad):
    mesh = plsc.VectorSubcoreMesh(core_axis_name="c", subcore_axis_name="s",
                                  num_cores=_NC, num_subcores=_NS)
    return pl.kernel(
        _sc_body,
        out_type=(
            jax.ShapeDtypeStruct((_BS, _S * _S), jnp.int32),
            jax.ShapeDtypeStruct((_BS, _S * _S), jnp.float32),
            jax.ShapeDtypeStruct((_BS, 4 * _S * _S), jnp.float32),
        ),
        mesh=mesh,
        compiler_params=pltpu.CompilerParams(needs_layout_passes=False,
                                             use_tc_tiling_on_sc=False),
        scratch_types=[
            pltpu.VMEM((512,), jnp.float32),
            pltpu.VMEM((_NPAD,), jnp.int32),
            pltpu.VMEM((_NPAD, 10), jnp.float32),
            pltpu.VMEM((_NPAD,), jnp.int32),
            pltpu.VMEM((_S * _S,), jnp.int32),
            pltpu.VMEM((_S * _S,), jnp.float32),
            pltpu.VMEM((4 * _S * _S,), jnp.float32),
            pltpu.SemaphoreType.DMA,
        ],
    )(pred_rows, tgt_pad)


_RB = 16


def _focal0(logits):
    bce = jnp.maximum(logits, 0.0) + jnp.log1p(jnp.exp(-jnp.abs(logits)))
    p_t = _sigmoid(-logits)
    return 0.75 * (1.0 - p_t) ** 2 * bce


def _softplus(v):
    return jnp.maximum(v, 0.0) + jnp.log1p(jnp.exp(-jnp.abs(v)))


def _tc_body(pred_ref, meta_ref, iou_ref, box_ref, out_ref):
    rb = pl.program_id(1)
    x = pred_ref[0]
    meta = meta_ref[0]
    tiou = iou_ref[0]
    box = box_ref[0]

    xt = jnp.transpose(x.reshape(_RB * _S, 90), (1, 0)).reshape(90, _RB, _S)

    obj = (meta & 1).astype(jnp.float32)
    resp = ((meta >> 1) & 1) == 1
    cid = meta >> 2

    cols_g = lax.broadcasted_iota(jnp.int32, (_RB, _S), 1).astype(jnp.float32)
    rows_g = (lax.broadcasted_iota(jnp.int32, (_RB, _S), 0)
              + rb * _RB).astype(jnp.float32)

    pr = [jnp.where(resp, xt[5 + k], xt[k]) for k in range(5)]

    pd_cx = jnp.clip((_sigmoid(pr[0]) + cols_g) * _CELL, 0.0, 1.0)
    pd_cy = jnp.clip((_sigmoid(pr[1]) + rows_g) * _CELL, 0.0, 1.0)
    pd_w = jnp.clip(jnp.exp(jnp.clip(pr[2], -10.0, 10.0)) * _CELL, 1e-6, 1.0)
    pd_h = jnp.clip(jnp.exp(jnp.clip(pr[3], -10.0, 10.0)) * _CELL, 1e-6, 1.0)
    gd_cx = (box[0] + cols_g) * _CELL
    gd_cy = (box[1] + rows_g) * _CELL
    gd_w = box[2]
    gd_h = box[3]

    eps = 1e-6
    pw = jnp.clip(pd_w, eps, None)
    ph = jnp.clip(pd_h, eps, None)
    gw = jnp.clip(gd_w, eps, None)
    gh = jnp.clip(gd_h, eps, None)
    px1 = pd_cx - pw * 0.5
    py1 = pd_cy - ph * 0.5
    px2 = pd_cx + pw * 0.5
    py2 = pd_cy + ph * 0.5
    gx1 = gd_cx - gw * 0.5
    gy1 = gd_cy - gh * 0.5
    gx2 = gd_cx + gw * 0.5
    gy2 = gd_cy + gh * 0.5
    ix1 = jnp.maximum(px1, gx1)
    iy1 = jnp.maximum(py1, gy1)
    ix2 = jnp.minimum(px2, gx2)
    iy2 = jnp.minimum(py2, gy2)
    inter = jnp.clip(ix2 - ix1, 0.0, None) * jnp.clip(iy2 - iy1, 0.0, None)
    union = (px2 - px1) * (py2 - py1) + (gx2 - gx1) * (gy2 - gy1) - inter
    iou_val = inter / (union + eps)
    rho2 = (pd_cx - gd_cx) ** 2 + (pd_cy - gd_cy) ** 2
    ex1 = jnp.minimum(px1, gx1)
    ey1 = jnp.minimum(py1, gy1)
    ex2 = jnp.maximum(px2, gx2)
    ey2 = jnp.maximum(py2, gy2)
    c2 = (ex2 - ex1) ** 2 + (ey2 - ey1) ** 2 + eps
    v = (4.0 / math.pi ** 2) * (lax.atan2(gw, gh + eps) - lax.atan2(pw, ph + eps)) ** 2
    alpha = v / (1.0 - iou_val + v + eps)
    ciou = 1.0 - iou_val + rho2 / c2 + alpha * v
    ciou = jnp.nan_to_num(ciou, nan=0.0, posinf=10.0, neginf=0.0)

    t = tiou
    logits = pr[4]
    bce = (jnp.maximum(logits, 0.0) - logits * t
           + jnp.log1p(jnp.exp(-jnp.abs(logits))))
    p_t = t * _sigmoid(logits) + (1.0 - t) * _sigmoid(-logits)
    a_t = t * 0.25 + (1.0 - t) * 0.75
    conf_f = a_t * (1.0 - p_t) ** 2 * bce

    noobj = _focal0(xt[4]) + _focal0(xt[9])

    ch_iota = lax.broadcasted_iota(jnp.int32, (90, _RB, _S), 0)
    sel = ch_iota == (cid + 10)[None]
    contrib = _softplus(xt) - jnp.where(sel, xt, 0.0)
    cls_all = jnp.sum(contrib, axis=0)
    box10 = sum(_softplus(xt[k]) for k in range(10))
    cls_term = cls_all - box10

    partial = jnp.sum(
        (_LC * ciou + conf_f + cls_term) * obj + _LN * noobj * (1.0 - obj)
    )

    @pl.when((pl.program_id(0) == 0) & (rb == 0))
    def _():
        out_ref[0, 0] = 0.0
    out_ref[0, 0] += partial


def _tc_loss(predictions, meta, iou, box):
    grid = (_BS, _S // _RB)
    return pl.pallas_call(
        _tc_body,
        grid=grid,
        in_specs=[
            pl.BlockSpec((1, _RB, _S, _Bb * 5 + _Cc), lambda b, r: (b, r, 0, 0)),
            pl.BlockSpec((1, _RB, _S), lambda b, r: (b, r, 0)),
            pl.BlockSpec((1, _RB, _S), lambda b, r: (b, r, 0)),
            pl.BlockSpec((1, 4, _RB, _S), lambda b, r: (b, 0, r, 0)),
        ],
        out_specs=pl.BlockSpec(memory_space=pltpu.SMEM,
                               block_shape=(1, 1),
                               index_map=lambda b, r: (0, 0)),
        out_shape=jax.ShapeDtypeStruct((1, 1), jnp.float32),
    )(predictions, meta, iou, box)


def kernel(predictions, targets):
    tgt_pad = jnp.zeros((_BS, 512), jnp.float32)
    tgt_pad = tgt_pad.at[:, : _NT * 5].set(targets.reshape(_BS, _NT * 5))
    pred_rows = predictions.reshape(_BS * _S * _S * 9, 10)
    meta, iou, box = _sc_build(pred_rows, tgt_pad)
    acc = _tc_loss(predictions,
                   meta.reshape(_BS, _S, _S),
                   iou.reshape(_BS, _S, _S),
                   box.reshape(_BS, 4, _S, _S))
    return acc[0, 0] / _BS


# speedup vs baseline: 16.5186x; 5.4504x over previous
import functools
import math

import jax
import jax.numpy as jnp
from jax import lax
from jax.experimental import pallas as pl
from jax.experimental.pallas import tpu as pltpu
from jax.experimental.pallas import tpu_sc as plsc

_Bb = 2
_Cc = 80
_LC = 5.0
_LN = 0.1
_BS = 32
_S = 128
_NT = 100
_CELL = 1.0 / _S
_NCHUNK = 7
_NPAD = _NCHUNK * 16

_NC = 2
_NS = 16
_L = 16


def _sigmoid(x):
    return 1.0 / (1.0 + jnp.exp(-x))


def _sc_body(pred_rows, tgt, meta_out, iou_out, box_out,
             tgt_v, idx_v, rows_v, mbuf_v, meta_v, iou_v, box_v, sem):
    b = lax.axis_index("s") * _NC + lax.axis_index("c")
    lanes = lax.iota(jnp.int32, _L)
    zf = jnp.zeros((_L,), jnp.float32)
    zi = jnp.zeros((_L,), jnp.int32)

    def _zero16(i, _):
        iou_v[pl.ds(i * _L, _L)] = zf
        meta_v[pl.ds(i * _L, _L)] = zi
        return 0
    lax.fori_loop(0, (_S * _S) // _L, _zero16, 0)

    def _zero_box(i, _):
        box_v[pl.ds(i * _L, _L)] = zf
        return 0
    lax.fori_loop(0, (4 * _S * _S) // _L, _zero_box, 0)

    pltpu.sync_copy(tgt.at[b], tgt_v)

    def _cells(c):
        lane = lanes + c * _L
        lv = jnp.minimum(lane, _NT - 1)
        base = lv * 5
        cx = plsc.load_gather(tgt_v, [base + 1])
        cy = plsc.load_gather(tgt_v, [base + 2])
        col_f = cx * float(_S)
        row_f = cy * float(_S)
        cols_c = jnp.clip(col_f.astype(jnp.int32), 0, _S - 1)
        rows_c = jnp.clip(row_f.astype(jnp.int32), 0, _S - 1)
        valid = lane < _NT
        return lane, lv, col_f, row_f, cols_c, rows_c, valid

    def _p1(c, _):
        lane, _, _, _, cols_c, rows_c, _ = _cells(c)
        r0 = (b * _S + rows_c) * 90 + ((cols_c * 90) >> 7)
        idx_v[0, pl.ds(c * _L, _L)] = r0
        idx_v[1, pl.ds(c * _L, _L)] = jnp.minimum(r0 + 1, _BS * _S * 90 - 1)
        return 0
    lax.fori_loop(0, _NCHUNK, _p1, 0)

    pltpu.async_copy(pred_rows.at[idx_v.at[0]], rows_v.at[pl.ds(0, _NPAD)], sem).wait()
    pltpu.async_copy(pred_rows.at[idx_v.at[1]], rows_v.at[pl.ds(_NPAD, _NPAD)], sem).wait()

    def _p2(c, _):
        lane, lv, col_f, row_f, cols_c, rows_c, valid = _cells(c)
        base = lv * 5
        cls_i = plsc.load_gather(tgt_v, [base + 0]).astype(jnp.int32)
        w = plsc.load_gather(tgt_v, [base + 3])
        h = plsc.load_gather(tgt_v, [base + 4])
        cx = col_f * _CELL
        cy = row_f * _CELL
        cx_rel = col_f - cols_c.astype(jnp.float32)
        cy_rel = row_f - rows_c.astype(jnp.float32)

        base_off = (cols_c * 90) & 127
        ch = []
        for k in range(10):
            off = base_off + k
            row_sel = jnp.where(off >= 128, lane + _NPAD, lane)
            ch.append(plsc.load_gather(rows_v, [row_sel, off & 127]))

        gx1 = cx - w * 0.5
        gy1 = cy - h * 0.5
        gx2 = cx + w * 0.5
        gy2 = cy + h * 0.5
        garea = jnp.clip(gx2 - gx1, 0.0, None) * jnp.clip(gy2 - gy1, 0.0, None)

        def _iou_one(tx, ty, tw, th):
            p_cx = jnp.clip((_sigmoid(tx) + cols_c.astype(jnp.float32)) * _CELL, 0.0, 1.0)
            p_cy = jnp.clip((_sigmoid(ty) + rows_c.astype(jnp.float32)) * _CELL, 0.0, 1.0)
            p_w = jnp.clip(jnp.exp(jnp.clip(tw, -10.0, 10.0)) * _CELL, 1e-6, 1.0)
            p_h = jnp.clip(jnp.exp(jnp.clip(th, -10.0, 10.0)) * _CELL, 1e-6, 1.0)
            px1 = p_cx - p_w * 0.5
            py1 = p_cy - p_h * 0.5
            px2 = p_cx + p_w * 0.5
            py2 = p_cy + p_h * 0.5
            ix1 = jnp.maximum(px1, gx1)
            iy1 = jnp.maximum(py1, gy1)
            ix2 = jnp.minimum(px2, gx2)
            iy2 = jnp.minimum(py2, gy2)
            inter = jnp.clip(ix2 - ix1, 0.0, None) * jnp.clip(iy2 - iy1, 0.0, None)
            union = (jnp.clip(px2 - px1, 0.0, None) * jnp.clip(py2 - py1, 0.0, None)
                     + garea - inter)
            return inter / (union + 1e-6)

        iou0 = _iou_one(ch[0], ch[1], ch[2], ch[3])
        iou1 = _iou_one(ch[5], ch[6], ch[7], ch[8])
        best_js = (iou1 > iou0).astype(jnp.int32)
        best_iou = jnp.maximum(iou0, iou1)
        mval = 1 | (best_js << 1) | (cls_i << 2)
        mbuf_v[pl.ds(c * _L, _L)] = mval

        cell_idx = rows_c * _S + cols_c
        plsc.store_scatter(iou_v, [cell_idx], best_iou, mask=valid)
        plsc.store_scatter(box_v, [cell_idx], cx_rel, mask=valid)
        plsc.store_scatter(box_v, [cell_idx + _S * _S], cy_rel, mask=valid)
        plsc.store_scatter(box_v, [cell_idx + 2 * _S * _S], w, mask=valid)
        plsc.store_scatter(box_v, [cell_idx + 3 * _S * _S], h, mask=valid)
        plsc.store_scatter(meta_v, [cell_idx], mval, mask=valid)
        return 0
    lax.fori_loop(0, _NCHUNK, _p2, 0)

    def _p3(c, _):
        _, _, col_f, _, cols_c, rows_c, valid = _cells(c)
        frac = col_f - cols_c.astype(jnp.float32)
        dx = jnp.where(frac >= 0.5, 1, -1).astype(jnp.int32)
        nc = cols_c + dx
        ok = valid & (nc >= 0) & (nc < _S)
        mval = mbuf_v[pl.ds(c * _L, _L)]
        plsc.store_scatter(meta_v, [rows_c * _S + jnp.clip(nc, 0, _S - 1)], mval, mask=ok)
        return 0
    lax.fori_loop(0, _NCHUNK, _p3, 0)

    def _p4(c, _):
        _, _, _, row_f, cols_c, rows_c, valid = _cells(c)
        frac = row_f - rows_c.astype(jnp.float32)
        dy = jnp.where(frac >= 0.5, 1, -1).astype(jnp.int32)
        nr = rows_c + dy
        ok = valid & (nr >= 0) & (nr < _S)
        mval = mbuf_v[pl.ds(c * _L, _L)]
        plsc.store_scatter(meta_v, [jnp.clip(nr, 0, _S - 1) * _S + cols_c], mval, mask=ok)
        return 0
    lax.fori_loop(0, _NCHUNK, _p4, 0)

    pltpu.sync_copy(meta_v, meta_out.at[b])
    pltpu.sync_copy(iou_v, iou_out.at[b])
    pltpu.sync_copy(box_v, box_out.at[b])


def _sc_build(pred_rows, tgt_pad):
    mesh = plsc.VectorSubcoreMesh(core_axis_name="c", subcore_axis_name="s",
                                  num_cores=_NC, num_subcores=_NS)
    return pl.kernel(
        _sc_body,
        out_type=(
            jax.ShapeDtypeStruct((_BS, _S * _S), jnp.int32),
            jax.ShapeDtypeStruct((_BS, _S * _S), jnp.float32),
            jax.ShapeDtypeStruct((_BS, 4 * _S * _S), jnp.float32),
        ),
        mesh=mesh,
        compiler_params=pltpu.CompilerParams(needs_layout_passes=False,
                                             use_tc_tiling_on_sc=False),
        scratch_types=[
            pltpu.VMEM((512,), jnp.float32),
            pltpu.VMEM((2, _NPAD), jnp.int32),
            pltpu.VMEM((2 * _NPAD, 128), jnp.float32),
            pltpu.VMEM((_NPAD,), jnp.int32),
            pltpu.VMEM((_S * _S,), jnp.int32),
            pltpu.VMEM((_S * _S,), jnp.float32),
            pltpu.VMEM((4 * _S * _S,), jnp.float32),
            pltpu.SemaphoreType.DMA,
        ],
    )(pred_rows, tgt_pad)


_RB = 16


def _focal0(logits):
    bce = jnp.maximum(logits, 0.0) + jnp.log1p(jnp.exp(-jnp.abs(logits)))
    p_t = _sigmoid(-logits)
    return 0.75 * (1.0 - p_t) ** 2 * bce


def _softplus(v):
    return jnp.maximum(v, 0.0) + jnp.log1p(jnp.exp(-jnp.abs(v)))


def _tc_body(pred_ref, meta_ref, iou_ref, box_ref, out_ref):
    rb = pl.program_id(1)
    x = pred_ref[0]
    meta = meta_ref[0]
    tiou = iou_ref[0]
    box = box_ref[0]

    xt = jnp.transpose(x.reshape(_RB * _S, 90), (1, 0)).reshape(90, _RB, _S)

    obj = (meta & 1).astype(jnp.float32)
    resp = ((meta >> 1) & 1) == 1
    cid = meta >> 2

    cols_g = lax.broadcasted_iota(jnp.int32, (_RB, _S), 1).astype(jnp.float32)
    rows_g = (lax.broadcasted_iota(jnp.int32, (_RB, _S), 0)
              + rb * _RB).astype(jnp.float32)

    pr = [jnp.where(resp, xt[5 + k], xt[k]) for k in range(5)]

    pd_cx = jnp.clip((_sigmoid(pr[0]) + cols_g) * _CELL, 0.0, 1.0)
    pd_cy = jnp.clip((_sigmoid(pr[1]) + rows_g) * _CELL, 0.0, 1.0)
    pd_w = jnp.clip(jnp.exp(jnp.clip(pr[2], -10.0, 10.0)) * _CELL, 1e-6, 1.0)
    pd_h = jnp.clip(jnp.exp(jnp.clip(pr[3], -10.0, 10.0)) * _CELL, 1e-6, 1.0)
    gd_cx = (box[0] + cols_g) * _CELL
    gd_cy = (box[1] + rows_g) * _CELL
    gd_w = box[2]
    gd_h = box[3]

    eps = 1e-6
    pw = jnp.clip(pd_w, eps, None)
    ph = jnp.clip(pd_h, eps, None)
    gw = jnp.clip(gd_w, eps, None)
    gh = jnp.clip(gd_h, eps, None)
    px1 = pd_cx - pw * 0.5
    py1 = pd_cy - ph * 0.5
    px2 = pd_cx + pw * 0.5
    py2 = pd_cy + ph * 0.5
    gx1 = gd_cx - gw * 0.5
    gy1 = gd_cy - gh * 0.5
    gx2 = gd_cx + gw * 0.5
    gy2 = gd_cy + gh * 0.5
    ix1 = jnp.maximum(px1, gx1)
    iy1 = jnp.maximum(py1, gy1)
    ix2 = jnp.minimum(px2, gx2)
    iy2 = jnp.minimum(py2, gy2)
    inter = jnp.clip(ix2 - ix1, 0.0, None) * jnp.clip(iy2 - iy1, 0.0, None)
    union = (px2 - px1) * (py2 - py1) + (gx2 - gx1) * (gy2 - gy1) - inter
    iou_val = inter / (union + eps)
    rho2 = (pd_cx - gd_cx) ** 2 + (pd_cy - gd_cy) ** 2
    ex1 = jnp.minimum(px1, gx1)
    ey1 = jnp.minimum(py1, gy1)
    ex2 = jnp.maximum(px2, gx2)
    ey2 = jnp.maximum(py2, gy2)
    c2 = (ex2 - ex1) ** 2 + (ey2 - ey1) ** 2 + eps
    v = (4.0 / math.pi ** 2) * (lax.atan2(gw, gh + eps) - lax.atan2(pw, ph + eps)) ** 2
    alpha = v / (1.0 - iou_val + v + eps)
    ciou = 1.0 - iou_val + rho2 / c2 + alpha * v
    ciou = jnp.nan_to_num(ciou, nan=0.0, posinf=10.0, neginf=0.0)

    t = tiou
    logits = pr[4]
    bce = (jnp.maximum(logits, 0.0) - logits * t
           + jnp.log1p(jnp.exp(-jnp.abs(logits))))
    p_t = t * _sigmoid(logits) + (1.0 - t) * _sigmoid(-logits)
    a_t = t * 0.25 + (1.0 - t) * 0.75
    conf_f = a_t * (1.0 - p_t) ** 2 * bce

    noobj = _focal0(xt[4]) + _focal0(xt[9])

    ch_iota = lax.broadcasted_iota(jnp.int32, (90, _RB, _S), 0)
    sel = ch_iota == (cid + 10)[None]
    contrib = _softplus(xt) - jnp.where(sel, xt, 0.0)
    cls_all = jnp.sum(contrib, axis=0)
    box10 = sum(_softplus(xt[k]) for k in range(10))
    cls_term = cls_all - box10

    partial = jnp.sum(
        (_LC * ciou + conf_f + cls_term) * obj + _LN * noobj * (1.0 - obj)
    )

    @pl.when((pl.program_id(0) == 0) & (rb == 0))
    def _():
        out_ref[0, 0] = 0.0
    out_ref[0, 0] += partial


def _tc_loss(predictions, meta, iou, box):
    grid = (_BS, _S // _RB)
    return pl.pallas_call(
        _tc_body,
        grid=grid,
        in_specs=[
            pl.BlockSpec((1, _RB, _S, _Bb * 5 + _Cc), lambda b, r: (b, r, 0, 0)),
            pl.BlockSpec((1, _RB, _S), lambda b, r: (b, r, 0)),
            pl.BlockSpec((1, _RB, _S), lambda b, r: (b, r, 0)),
            pl.BlockSpec((1, 4, _RB, _S), lambda b, r: (b, 0, r, 0)),
        ],
        out_specs=pl.BlockSpec(memory_space=pltpu.SMEM,
                               block_shape=(1, 1),
                               index_map=lambda b, r: (0, 0)),
        out_shape=jax.ShapeDtypeStruct((1, 1), jnp.float32),
    )(predictions, meta, iou, box)


def kernel(predictions, targets):
    tgt_pad = jnp.zeros((_BS, 512), jnp.float32)
    tgt_pad = tgt_pad.at[:, : _NT * 5].set(targets.reshape(_BS, _NT * 5))
    pred_rows = predictions.reshape(_BS, _S, _S * 90).reshape(_BS * _S * 90, 128)
    meta, iou, box = _sc_build(pred_rows, tgt_pad)
    acc = _tc_loss(predictions,
                   meta.reshape(_BS, _S, _S),
                   iou.reshape(_BS, _S, _S),
                   box.reshape(_BS, 4, _S, _S))
    return acc[0, 0] / _BS


# speedup vs baseline: 30.1159x; 1.8232x over previous
import functools
import math

import jax
import jax.numpy as jnp
from jax import lax
from jax.experimental import pallas as pl
from jax.experimental.pallas import tpu as pltpu
from jax.experimental.pallas import tpu_sc as plsc

_Bb = 2
_Cc = 80
_LC = 5.0
_LN = 0.1
_BS = 32
_S = 128
_NT = 100
_CELL = 1.0 / _S
_NCHUNK = 7
_NPAD = _NCHUNK * 16

_NC = 2
_NS = 16
_L = 16


def _sigmoid(x):
    return 1.0 / (1.0 + jnp.exp(-x))


def _mesh():
    return plsc.VectorSubcoreMesh(core_axis_name="c", subcore_axis_name="s",
                                  num_cores=_NC, num_subcores=_NS)


_SC_PARAMS = pltpu.CompilerParams(needs_layout_passes=False,
                                  use_tc_tiling_on_sc=False)


def _cells_of(tgt_v, c):
    lanes = lax.iota(jnp.int32, _L)
    lane = lanes + c * _L
    lv = jnp.minimum(lane, _NT - 1)
    base = lv * 5
    cx = plsc.load_gather(tgt_v, [base + 1])
    cy = plsc.load_gather(tgt_v, [base + 2])
    col_f = cx * float(_S)
    row_f = cy * float(_S)
    cols_c = jnp.clip(col_f.astype(jnp.int32), 0, _S - 1)
    rows_c = jnp.clip(row_f.astype(jnp.int32), 0, _S - 1)
    valid = lane < _NT
    return lane, lv, col_f, row_f, cols_c, rows_c, valid


def _sc1_body(tgt, meta_out, box_out, tgt_v, mbuf_v, meta_v, box_v):
    b = lax.axis_index("s") * _NC + lax.axis_index("c")
    zf = jnp.zeros((_L,), jnp.float32)
    zi = jnp.zeros((_L,), jnp.int32)

    def _zero_m(i, _):
        meta_v[pl.ds(i * _L, _L)] = zi
        return 0
    lax.fori_loop(0, (_S * _S) // _L, _zero_m, 0)

    def _zero_b(i, _):
        box_v[pl.ds(i * _L, _L)] = zf
        return 0
    lax.fori_loop(0, (6 * _S * _S) // _L, _zero_b, 0)

    pltpu.sync_copy(tgt.at[b], tgt_v)

    def _pc(c, _):
        lane, lv, col_f, row_f, cols_c, rows_c, valid = _cells_of(tgt_v, c)
        base = lv * 5
        cls_i = plsc.load_gather(tgt_v, [base + 0]).astype(jnp.int32)
        cx = plsc.load_gather(tgt_v, [base + 1])
        cy = plsc.load_gather(tgt_v, [base + 2])
        w = plsc.load_gather(tgt_v, [base + 3])
        h = plsc.load_gather(tgt_v, [base + 4])
        cx_rel = col_f - cols_c.astype(jnp.float32)
        cy_rel = row_f - rows_c.astype(jnp.float32)
        mval = 1 | (cls_i << 1)
        mbuf_v[pl.ds(c * _L, _L)] = mval
        cell = rows_c * _S + cols_c
        plsc.store_scatter(meta_v, [cell], mval, mask=valid)
        plsc.store_scatter(box_v, [cell], cx, mask=valid)
        plsc.store_scatter(box_v, [cell + _S * _S], cy, mask=valid)
        plsc.store_scatter(box_v, [cell + 2 * _S * _S], w, mask=valid)
        plsc.store_scatter(box_v, [cell + 3 * _S * _S], h, mask=valid)
        plsc.store_scatter(box_v, [cell + 4 * _S * _S], cx_rel, mask=valid)
        plsc.store_scatter(box_v, [cell + 5 * _S * _S], cy_rel, mask=valid)
        return 0
    lax.fori_loop(0, _NCHUNK, _pc, 0)

    def _pnx(c, _):
        _, _, col_f, _, cols_c, rows_c, valid = _cells_of(tgt_v, c)
        frac = col_f - cols_c.astype(jnp.float32)
        dx = jnp.where(frac >= 0.5, 1, -1).astype(jnp.int32)
        nc = cols_c + dx
        ok = valid & (nc >= 0) & (nc < _S)
        mval = mbuf_v[pl.ds(c * _L, _L)]
        plsc.store_scatter(meta_v, [rows_c * _S + jnp.clip(nc, 0, _S - 1)], mval, mask=ok)
        return 0
    lax.fori_loop(0, _NCHUNK, _pnx, 0)

    def _pny(c, _):
        _, _, _, row_f, cols_c, rows_c, valid = _cells_of(tgt_v, c)
        frac = row_f - rows_c.astype(jnp.float32)
        dy = jnp.where(frac >= 0.5, 1, -1).astype(jnp.int32)
        nr = rows_c + dy
        ok = valid & (nr >= 0) & (nr < _S)
        mval = mbuf_v[pl.ds(c * _L, _L)]
        plsc.store_scatter(meta_v, [jnp.clip(nr, 0, _S - 1) * _S + cols_c], mval, mask=ok)
        return 0
    lax.fori_loop(0, _NCHUNK, _pny, 0)

    pltpu.sync_copy(meta_v, meta_out.at[b])
    pltpu.sync_copy(box_v, box_out.at[b])


def _sc1(tgt_pad):
    return pl.kernel(
        _sc1_body,
        out_type=(
            jax.ShapeDtypeStruct((_BS, _S * _S), jnp.int32),
            jax.ShapeDtypeStruct((_BS, 6 * _S * _S), jnp.float32),
        ),
        mesh=_mesh(),
        compiler_params=_SC_PARAMS,
        scratch_types=[
            pltpu.VMEM((512,), jnp.float32),
            pltpu.VMEM((_NPAD,), jnp.int32),
            pltpu.VMEM((_S * _S,), jnp.int32),
            pltpu.VMEM((6 * _S * _S,), jnp.float32),
        ],
    )(tgt_pad)


def _sc2_body(tgt, bjs_rows, meta_out, tgt_v, idx_v, rows_v, mbuf_v, meta_v, sem):
    b = lax.axis_index("s") * _NC + lax.axis_index("c")
    zi = jnp.zeros((_L,), jnp.int32)

    def _zero_m(i, _):
        meta_v[pl.ds(i * _L, _L)] = zi
        return 0
    lax.fori_loop(0, (_S * _S) // _L, _zero_m, 0)

    pltpu.sync_copy(tgt.at[b], tgt_v)

    def _p1(c, _):
        _, _, _, _, cols_c, rows_c, _ = _cells_of(tgt_v, c)
        g = b * (_S * _S) + rows_c * _S + cols_c
        idx_v[0, pl.ds(c * _L, _L)] = g >> 7
        return 0
    lax.fori_loop(0, _NCHUNK, _p1, 0)

    pltpu.async_copy(bjs_rows.at[idx_v.at[0]], rows_v, sem).wait()

    def _p2(c, _):
        lane, _, _, _, cols_c, rows_c, valid = _cells_of(tgt_v, c)
        g = b * (_S * _S) + rows_c * _S + cols_c
        bjs = plsc.load_gather(rows_v, [lane, g & 127])
        mval = 1 | (bjs << 1)
        mbuf_v[pl.ds(c * _L, _L)] = mval
        plsc.store_scatter(meta_v, [rows_c * _S + cols_c], mval, mask=valid)
        return 0
    lax.fori_loop(0, _NCHUNK, _p2, 0)

    def _pnx(c, _):
        _, _, col_f, _, cols_c, rows_c, valid = _cells_of(tgt_v, c)
        frac = col_f - cols_c.astype(jnp.float32)
        dx = jnp.where(frac >= 0.5, 1, -1).astype(jnp.int32)
        nc = cols_c + dx
        ok = valid & (nc >= 0) & (nc < _S)
        mval = mbuf_v[pl.ds(c * _L, _L)]
        plsc.store_scatter(meta_v, [rows_c * _S + jnp.clip(nc, 0, _S - 1)], mval, mask=ok)
        return 0
    lax.fori_loop(0, _NCHUNK, _pnx, 0)

    def _pny(c, _):
        _, _, _, row_f, cols_c, rows_c, valid = _cells_of(tgt_v, c)
        frac = row_f - rows_c.astype(jnp.float32)
        dy = jnp.where(frac >= 0.5, 1, -1).astype(jnp.int32)
        nr = rows_c + dy
        ok = valid & (nr >= 0) & (nr < _S)
        mval = mbuf_v[pl.ds(c * _L, _L)]
        plsc.store_scatter(meta_v, [jnp.clip(nr, 0, _S - 1) * _S + cols_c], mval, mask=ok)
        return 0
    lax.fori_loop(0, _NCHUNK, _pny, 0)

    pltpu.sync_copy(meta_v, meta_out.at[b])


def _sc2(tgt_pad, bjs_rows):
    return pl.kernel(
        _sc2_body,
        out_type=jax.ShapeDtypeStruct((_BS, _S * _S), jnp.int32),
        mesh=_mesh(),
        compiler_params=_SC_PARAMS,
        scratch_types=[
            pltpu.VMEM((512,), jnp.float32),
            pltpu.VMEM((1, _NPAD), jnp.int32),
            pltpu.VMEM((_NPAD, 128), jnp.int32),
            pltpu.VMEM((_NPAD,), jnp.int32),
            pltpu.VMEM((_S * _S,), jnp.int32),
            pltpu.SemaphoreType.DMA,
        ],
    )(tgt_pad, bjs_rows)


_RB = 64


def _softplus(v):
    return jnp.maximum(v, 0.0) + jnp.log1p(jnp.exp(-jnp.abs(v)))


def _focal(logits, t):
    bce = (jnp.maximum(logits, 0.0) - logits * t
           + jnp.log1p(jnp.exp(-jnp.abs(logits))))
    p_t = t * _sigmoid(logits) + (1.0 - t) * _sigmoid(-logits)
    a_t = t * 0.25 + (1.0 - t) * 0.75
    return a_t * (1.0 - p_t) ** 2 * bce


def _tc1_body(pred_ref, meta_ref, box_ref, bjs_ref, c0_ref, c1_ref,
              f0_ref, f1_ref, cls_ref, no_ref):
    rb = pl.program_id(1)
    x = pred_ref[0]
    meta = meta_ref[0]
    box = box_ref[0]

    xt = jnp.transpose(x.reshape(_RB * _S, 90), (1, 0)).reshape(90, _RB, _S)

    cid = meta >> 1

    cols_g = lax.broadcasted_iota(jnp.int32, (_RB, _S), 1).astype(jnp.float32)
    rows_g = (lax.broadcasted_iota(jnp.int32, (_RB, _S), 0)
              + rb * _RB).astype(jnp.float32)

    gax, gay, gw0, gh0, grx, gry = (box[k] for k in range(6))
    center = gw0 > 0.0
    gx1 = gax - gw0 * 0.5
    gy1 = gay - gh0 * 0.5
    gx2 = gax + gw0 * 0.5
    gy2 = gay + gh0 * 0.5
    garea = jnp.clip(gx2 - gx1, 0.0, None) * jnp.clip(gy2 - gy1, 0.0, None)

    pdec = []
    ious = []
    for j in range(2):
        p_cx = jnp.clip((_sigmoid(xt[5 * j + 0]) + cols_g) * _CELL, 0.0, 1.0)
        p_cy = jnp.clip((_sigmoid(xt[5 * j + 1]) + rows_g) * _CELL, 0.0, 1.0)
        p_w = jnp.clip(jnp.exp(jnp.clip(xt[5 * j + 2], -10.0, 10.0)) * _CELL, 1e-6, 1.0)
        p_h = jnp.clip(jnp.exp(jnp.clip(xt[5 * j + 3], -10.0, 10.0)) * _CELL, 1e-6, 1.0)
        pdec.append((p_cx, p_cy, p_w, p_h))
        px1 = p_cx - p_w * 0.5
        py1 = p_cy - p_h * 0.5
        px2 = p_cx + p_w * 0.5
        py2 = p_cy + p_h * 0.5
        ix1 = jnp.maximum(px1, gx1)
        iy1 = jnp.maximum(py1, gy1)
        ix2 = jnp.minimum(px2, gx2)
        iy2 = jnp.minimum(py2, gy2)
        inter = jnp.clip(ix2 - ix1, 0.0, None) * jnp.clip(iy2 - iy1, 0.0, None)
        union = (jnp.clip(px2 - px1, 0.0, None) * jnp.clip(py2 - py1, 0.0, None)
                 + garea - inter)
        ious.append(inter / (union + 1e-6))

    bjs_ref[0] = (ious[1] > ious[0]).astype(jnp.int32)
    t = jnp.where(center, jnp.maximum(ious[0], ious[1]), 0.0)

    gd_cx = (grx + cols_g) * _CELL
    gd_cy = (gry + rows_g) * _CELL

    eps = 1e-6
    gw = jnp.clip(gw0, eps, None)
    gh = jnp.clip(gh0, eps, None)
    ggx1 = gd_cx - gw * 0.5
    ggy1 = gd_cy - gh * 0.5
    ggx2 = gd_cx + gw * 0.5
    ggy2 = gd_cy + gh * 0.5
    atan_g = lax.atan2(gw, gh + eps)
    area_g = (ggx2 - ggx1) * (ggy2 - ggy1)

    for j, out_ref in ((0, c0_ref), (1, c1_ref)):
        pd_cx, pd_cy, pd_w, pd_h = pdec[j]
        pw = jnp.clip(pd_w, eps, None)
        ph = jnp.clip(pd_h, eps, None)
        px1 = pd_cx - pw * 0.5
        py1 = pd_cy - ph * 0.5
        px2 = pd_cx + pw * 0.5
        py2 = pd_cy + ph * 0.5
        ix1 = jnp.maximum(px1, ggx1)
        iy1 = jnp.maximum(py1, ggy1)
        ix2 = jnp.minimum(px2, ggx2)
        iy2 = jnp.minimum(py2, ggy2)
        inter = jnp.clip(ix2 - ix1, 0.0, None) * jnp.clip(iy2 - iy1, 0.0, None)
        union = (px2 - px1) * (py2 - py1) + area_g - inter
        iou_val = inter / (union + eps)
        rho2 = (pd_cx - gd_cx) ** 2 + (pd_cy - gd_cy) ** 2
        ex1 = jnp.minimum(px1, ggx1)
        ey1 = jnp.minimum(py1, ggy1)
        ex2 = jnp.maximum(px2, ggx2)
        ey2 = jnp.maximum(py2, ggy2)
        c2 = (ex2 - ex1) ** 2 + (ey2 - ey1) ** 2 + eps
        v = (4.0 / math.pi ** 2) * (atan_g - lax.atan2(pw, ph + eps)) ** 2
        alpha = v / (1.0 - iou_val + v + eps)
        ciou = 1.0 - iou_val + rho2 / c2 + alpha * v
        out_ref[0] = jnp.nan_to_num(ciou, nan=0.0, posinf=10.0, neginf=0.0)

    f0_ref[0] = _focal(xt[4], t)
    f1_ref[0] = _focal(xt[9], t)
    no_ref[0] = (0.75 * _sigmoid(xt[4]) ** 2 * _softplus(xt[4])
                 + 0.75 * _sigmoid(xt[9]) ** 2 * _softplus(xt[9]))

    ch_iota = lax.broadcasted_iota(jnp.int32, (90, _RB, _S), 0)
    sel = ch_iota == (cid + 10)[None]
    contrib = _softplus(xt) - jnp.where(sel, xt, 0.0)
    cls_all = jnp.sum(contrib, axis=0)
    box10 = sum(_softplus(xt[k]) for k in range(10))
    cls_ref[0] = cls_all - box10


def _tc1(predictions, meta1, box6):
    grid = (_BS, _S // _RB)
    pf = jax.ShapeDtypeStruct((_BS, _S, _S), jnp.float32)
    blk = pl.BlockSpec((1, _RB, _S), lambda b, r: (b, r, 0))
    return pl.pallas_call(
        _tc1_body,
        grid=grid,
        in_specs=[
            pl.BlockSpec((1, _RB, _S, _Bb * 5 + _Cc), lambda b, r: (b, r, 0, 0)),
            blk,
            pl.BlockSpec((1, 6, _RB, _S), lambda b, r: (b, 0, r, 0)),
        ],
        out_specs=[blk] * 7,
        out_shape=[jax.ShapeDtypeStruct((_BS, _S, _S), jnp.int32),
                   pf, pf, pf, pf, pf, pf],
    )(predictions, meta1, box6)


_RB2 = 64


def _tc2_body(meta_ref, c0_ref, c1_ref, f0_ref, f1_ref, cls_ref, no_ref, out_ref):
    meta = meta_ref[0]
    obj = (meta & 1).astype(jnp.float32)
    resp = ((meta >> 1) & 1) == 1
    ciou = jnp.where(resp, c1_ref[0], c0_ref[0])
    fc = jnp.where(resp, f1_ref[0], f0_ref[0])
    partial = jnp.sum((_LC * ciou + fc + cls_ref[0]) * obj
                      + _LN * no_ref[0] * (1.0 - obj))

    @pl.when((pl.program_id(0) == 0) & (pl.program_id(1) == 0))
    def _():
        out_ref[0, 0] = 0.0
    out_ref[0, 0] += partial


def _tc2(meta2, c0, c1, f0, f1, clsm, nom):
    grid = (_BS, _S // _RB2)
    blk = pl.BlockSpec((1, _RB2, _S), lambda b, r: (b, r, 0))
    return pl.pallas_call(
        _tc2_body,
        grid=grid,
        in_specs=[blk] * 7,
        out_specs=pl.BlockSpec(memory_space=pltpu.SMEM,
                               block_shape=(1, 1),
                               index_map=lambda b, r: (0, 0)),
        out_shape=jax.ShapeDtypeStruct((1, 1), jnp.float32),
    )(meta2, c0, c1, f0, f1, clsm, nom)


def kernel(predictions, targets):
    tgt_pad = jnp.zeros((_BS, 512), jnp.float32)
    tgt_pad = tgt_pad.at[:, : _NT * 5].set(targets.reshape(_BS, _NT * 5))
    meta1, box6 = _sc1(tgt_pad)
    bjs, c0, c1, f0, f1, clsm, nom = _tc1(
        predictions,
        meta1.reshape(_BS, _S, _S),
        box6.reshape(_BS, 6, _S, _S),
    )
    meta2 = _sc2(tgt_pad, bjs.reshape(_BS * _S, _S))
    acc = _tc2(meta2.reshape(_BS, _S, _S), c0, c1, f0, f1, clsm, nom)
    return acc[0, 0] / _BS


# speedup vs baseline: 30.4676x; 1.0117x over previous
import functools
import math

import jax
import jax.numpy as jnp
from jax import lax
from jax.experimental import pallas as pl
from jax.experimental.pallas import tpu as pltpu
from jax.experimental.pallas import tpu_sc as plsc

_Bb = 2
_Cc = 80
_LC = 5.0
_LN = 0.1
_BS = 32
_S = 128
_NT = 100
_CELL = 1.0 / _S
_NCHUNK = 7
_NPAD = _NCHUNK * 16

_NC = 2
_NS = 16
_L = 16


def _sigmoid(x):
    return 1.0 / (1.0 + jnp.exp(-x))


def _mesh():
    return plsc.VectorSubcoreMesh(core_axis_name="c", subcore_axis_name="s",
                                  num_cores=_NC, num_subcores=_NS)


_SC_PARAMS = pltpu.CompilerParams(needs_layout_passes=False,
                                  use_tc_tiling_on_sc=False)


def _cells_of(tgt_v, c):
    lanes = lax.iota(jnp.int32, _L)
    lane = lanes + c * _L
    lv = jnp.minimum(lane, _NT - 1)
    base = lv * 5
    cx = plsc.load_gather(tgt_v, [base + 1])
    cy = plsc.load_gather(tgt_v, [base + 2])
    col_f = cx * float(_S)
    row_f = cy * float(_S)
    cols_c = jnp.clip(col_f.astype(jnp.int32), 0, _S - 1)
    rows_c = jnp.clip(row_f.astype(jnp.int32), 0, _S - 1)
    valid = lane < _NT
    return lane, lv, col_f, row_f, cols_c, rows_c, valid


def _sc1_body(tgt, meta_out, box_out, tgt_v, mbuf_v, meta_v, box_v):
    b = lax.axis_index("s") * _NC + lax.axis_index("c")
    zf = jnp.zeros((_L,), jnp.float32)
    zi = jnp.zeros((_L,), jnp.int32)

    def _zero_m(i, _):
        meta_v[pl.ds(i * _L, _L)] = zi
        return 0
    lax.fori_loop(0, (_S * _S) // _L, _zero_m, 0)

    def _zero_b(i, _):
        box_v[pl.ds(i * _L, _L)] = zf
        return 0
    lax.fori_loop(0, (6 * _S * _S) // _L, _zero_b, 0)

    pltpu.sync_copy(tgt.at[b], tgt_v)

    def _pc(c, _):
        lane, lv, col_f, row_f, cols_c, rows_c, valid = _cells_of(tgt_v, c)
        base = lv * 5
        cls_i = plsc.load_gather(tgt_v, [base + 0]).astype(jnp.int32)
        cx = plsc.load_gather(tgt_v, [base + 1])
        cy = plsc.load_gather(tgt_v, [base + 2])
        w = plsc.load_gather(tgt_v, [base + 3])
        h = plsc.load_gather(tgt_v, [base + 4])
        cx_rel = col_f - cols_c.astype(jnp.float32)
        cy_rel = row_f - rows_c.astype(jnp.float32)
        mval = 1 | (cls_i << 1)
        mbuf_v[pl.ds(c * _L, _L)] = mval
        cell = rows_c * _S + cols_c
        plsc.store_scatter(meta_v, [cell], mval, mask=valid)
        plsc.store_scatter(box_v, [cell], cx, mask=valid)
        plsc.store_scatter(box_v, [cell + _S * _S], cy, mask=valid)
        plsc.store_scatter(box_v, [cell + 2 * _S * _S], w, mask=valid)
        plsc.store_scatter(box_v, [cell + 3 * _S * _S], h, mask=valid)
        plsc.store_scatter(box_v, [cell + 4 * _S * _S], cx_rel, mask=valid)
        plsc.store_scatter(box_v, [cell + 5 * _S * _S], cy_rel, mask=valid)
        return 0
    lax.fori_loop(0, _NCHUNK, _pc, 0)

    def _pnx(c, _):
        _, _, col_f, _, cols_c, rows_c, valid = _cells_of(tgt_v, c)
        frac = col_f - cols_c.astype(jnp.float32)
        dx = jnp.where(frac >= 0.5, 1, -1).astype(jnp.int32)
        nc = cols_c + dx
        ok = valid & (nc >= 0) & (nc < _S)
        mval = mbuf_v[pl.ds(c * _L, _L)]
        plsc.store_scatter(meta_v, [rows_c * _S + jnp.clip(nc, 0, _S - 1)], mval, mask=ok)
        return 0
    lax.fori_loop(0, _NCHUNK, _pnx, 0)

    def _pny(c, _):
        _, _, _, row_f, cols_c, rows_c, valid = _cells_of(tgt_v, c)
        frac = row_f - rows_c.astype(jnp.float32)
        dy = jnp.where(frac >= 0.5, 1, -1).astype(jnp.int32)
        nr = rows_c + dy
        ok = valid & (nr >= 0) & (nr < _S)
        mval = mbuf_v[pl.ds(c * _L, _L)]
        plsc.store_scatter(meta_v, [jnp.clip(nr, 0, _S - 1) * _S + cols_c], mval, mask=ok)
        return 0
    lax.fori_loop(0, _NCHUNK, _pny, 0)

    pltpu.sync_copy(meta_v, meta_out.at[b])
    pltpu.sync_copy(box_v, box_out.at[b])


def _sc1(tgt_pad):
    return pl.kernel(
        _sc1_body,
        out_type=(
            jax.ShapeDtypeStruct((_BS, _S * _S), jnp.int32),
            jax.ShapeDtypeStruct((_BS, 6 * _S * _S), jnp.float32),
        ),
        mesh=_mesh(),
        compiler_params=_SC_PARAMS,
        scratch_types=[
            pltpu.VMEM((512,), jnp.float32),
            pltpu.VMEM((_NPAD,), jnp.int32),
            pltpu.VMEM((_S * _S,), jnp.int32),
            pltpu.VMEM((6 * _S * _S,), jnp.float32),
        ],
    )(tgt_pad)


def _sc2_body(tgt, bjs_rows, meta_out, tgt_v, idx_v, rows_v, mbuf_v, meta_v, sem):
    b = lax.axis_index("s") * _NC + lax.axis_index("c")
    zi = jnp.zeros((_L,), jnp.int32)

    def _zero_m(i, _):
        meta_v[pl.ds(i * _L, _L)] = zi
        return 0
    lax.fori_loop(0, (_S * _S) // _L, _zero_m, 0)

    pltpu.sync_copy(tgt.at[b], tgt_v)

    def _p1(c, _):
        _, _, _, _, cols_c, rows_c, _ = _cells_of(tgt_v, c)
        g = b * (_S * _S) + rows_c * _S + cols_c
        idx_v[0, pl.ds(c * _L, _L)] = g >> 7
        return 0
    lax.fori_loop(0, _NCHUNK, _p1, 0)

    pltpu.async_copy(bjs_rows.at[idx_v.at[0]], rows_v, sem).wait()

    def _p2(c, _):
        lane, _, _, _, cols_c, rows_c, valid = _cells_of(tgt_v, c)
        g = b * (_S * _S) + rows_c * _S + cols_c
        bjs = plsc.load_gather(rows_v, [lane, g & 127])
        mval = 1 | (bjs << 1)
        mbuf_v[pl.ds(c * _L, _L)] = mval
        plsc.store_scatter(meta_v, [rows_c * _S + cols_c], mval, mask=valid)
        return 0
    lax.fori_loop(0, _NCHUNK, _p2, 0)

    def _pnx(c, _):
        _, _, col_f, _, cols_c, rows_c, valid = _cells_of(tgt_v, c)
        frac = col_f - cols_c.astype(jnp.float32)
        dx = jnp.where(frac >= 0.5, 1, -1).astype(jnp.int32)
        nc = cols_c + dx
        ok = valid & (nc >= 0) & (nc < _S)
        mval = mbuf_v[pl.ds(c * _L, _L)]
        plsc.store_scatter(meta_v, [rows_c * _S + jnp.clip(nc, 0, _S - 1)], mval, mask=ok)
        return 0
    lax.fori_loop(0, _NCHUNK, _pnx, 0)

    def _pny(c, _):
        _, _, _, row_f, cols_c, rows_c, valid = _cells_of(tgt_v, c)
        frac = row_f - rows_c.astype(jnp.float32)
        dy = jnp.where(frac >= 0.5, 1, -1).astype(jnp.int32)
        nr = rows_c + dy
        ok = valid & (nr >= 0) & (nr < _S)
        mval = mbuf_v[pl.ds(c * _L, _L)]
        plsc.store_scatter(meta_v, [jnp.clip(nr, 0, _S - 1) * _S + cols_c], mval, mask=ok)
        return 0
    lax.fori_loop(0, _NCHUNK, _pny, 0)

    pltpu.sync_copy(meta_v, meta_out.at[b])


def _sc2(tgt_pad, bjs_rows):
    return pl.kernel(
        _sc2_body,
        out_type=jax.ShapeDtypeStruct((_BS, _S * _S), jnp.int32),
        mesh=_mesh(),
        compiler_params=_SC_PARAMS,
        scratch_types=[
            pltpu.VMEM((512,), jnp.float32),
            pltpu.VMEM((1, _NPAD), jnp.int32),
            pltpu.VMEM((_NPAD, 128), jnp.int32),
            pltpu.VMEM((_NPAD,), jnp.int32),
            pltpu.VMEM((_S * _S,), jnp.int32),
            pltpu.SemaphoreType.DMA,
        ],
    )(tgt_pad, bjs_rows)


_RB = 128


def _softplus(v):
    return jnp.maximum(v, 0.0) + jnp.log1p(jnp.exp(-jnp.abs(v)))


def _focal(logits, t):
    bce = (jnp.maximum(logits, 0.0) - logits * t
           + jnp.log1p(jnp.exp(-jnp.abs(logits))))
    p_t = t * _sigmoid(logits) + (1.0 - t) * _sigmoid(-logits)
    a_t = t * 0.25 + (1.0 - t) * 0.75
    return a_t * (1.0 - p_t) ** 2 * bce


def _tc1_body(pred_ref, meta_ref, box_ref, bjs_ref, c0_ref, c1_ref,
              f0_ref, f1_ref, cls_ref, no_ref):
    rb = pl.program_id(1)
    x = pred_ref[0]
    meta = meta_ref[0]
    box = box_ref[0]

    xt = jnp.transpose(x.reshape(_RB * _S, 90), (1, 0)).reshape(90, _RB, _S)

    cid = meta >> 1

    cols_g = lax.broadcasted_iota(jnp.int32, (_RB, _S), 1).astype(jnp.float32)
    rows_g = (lax.broadcasted_iota(jnp.int32, (_RB, _S), 0)
              + rb * _RB).astype(jnp.float32)

    gax, gay, gw0, gh0, grx, gry = (box[k] for k in range(6))
    center = gw0 > 0.0
    gx1 = gax - gw0 * 0.5
    gy1 = gay - gh0 * 0.5
    gx2 = gax + gw0 * 0.5
    gy2 = gay + gh0 * 0.5
    garea = jnp.clip(gx2 - gx1, 0.0, None) * jnp.clip(gy2 - gy1, 0.0, None)

    pdec = []
    ious = []
    for j in range(2):
        p_cx = jnp.clip((_sigmoid(xt[5 * j + 0]) + cols_g) * _CELL, 0.0, 1.0)
        p_cy = jnp.clip((_sigmoid(xt[5 * j + 1]) + rows_g) * _CELL, 0.0, 1.0)
        p_w = jnp.clip(jnp.exp(jnp.clip(xt[5 * j + 2], -10.0, 10.0)) * _CELL, 1e-6, 1.0)
        p_h = jnp.clip(jnp.exp(jnp.clip(xt[5 * j + 3], -10.0, 10.0)) * _CELL, 1e-6, 1.0)
        pdec.append((p_cx, p_cy, p_w, p_h))
        px1 = p_cx - p_w * 0.5
        py1 = p_cy - p_h * 0.5
        px2 = p_cx + p_w * 0.5
        py2 = p_cy + p_h * 0.5
        ix1 = jnp.maximum(px1, gx1)
        iy1 = jnp.maximum(py1, gy1)
        ix2 = jnp.minimum(px2, gx2)
        iy2 = jnp.minimum(py2, gy2)
        inter = jnp.clip(ix2 - ix1, 0.0, None) * jnp.clip(iy2 - iy1, 0.0, None)
        union = (jnp.clip(px2 - px1, 0.0, None) * jnp.clip(py2 - py1, 0.0, None)
                 + garea - inter)
        ious.append(inter / (union + 1e-6))

    bjs_ref[0] = (ious[1] > ious[0]).astype(jnp.int32)
    t = jnp.where(center, jnp.maximum(ious[0], ious[1]), 0.0)

    gd_cx = (grx + cols_g) * _CELL
    gd_cy = (gry + rows_g) * _CELL

    eps = 1e-6
    gw = jnp.clip(gw0, eps, None)
    gh = jnp.clip(gh0, eps, None)
    ggx1 = gd_cx - gw * 0.5
    ggy1 = gd_cy - gh * 0.5
    ggx2 = gd_cx + gw * 0.5
    ggy2 = gd_cy + gh * 0.5
    atan_g = lax.atan2(gw, gh + eps)
    area_g = (ggx2 - ggx1) * (ggy2 - ggy1)

    for j, out_ref in ((0, c0_ref), (1, c1_ref)):
        pd_cx, pd_cy, pd_w, pd_h = pdec[j]
        pw = jnp.clip(pd_w, eps, None)
        ph = jnp.clip(pd_h, eps, None)
        px1 = pd_cx - pw * 0.5
        py1 = pd_cy - ph * 0.5
        px2 = pd_cx + pw * 0.5
        py2 = pd_cy + ph * 0.5
        ix1 = jnp.maximum(px1, ggx1)
        iy1 = jnp.maximum(py1, ggy1)
        ix2 = jnp.minimum(px2, ggx2)
        iy2 = jnp.minimum(py2, ggy2)
        inter = jnp.clip(ix2 - ix1, 0.0, None) * jnp.clip(iy2 - iy1, 0.0, None)
        union = (px2 - px1) * (py2 - py1) + area_g - inter
        iou_val = inter / (union + eps)
        rho2 = (pd_cx - gd_cx) ** 2 + (pd_cy - gd_cy) ** 2
        ex1 = jnp.minimum(px1, ggx1)
        ey1 = jnp.minimum(py1, ggy1)
        ex2 = jnp.maximum(px2, ggx2)
        ey2 = jnp.maximum(py2, ggy2)
        c2 = (ex2 - ex1) ** 2 + (ey2 - ey1) ** 2 + eps
        v = (4.0 / math.pi ** 2) * (atan_g - lax.atan2(pw, ph + eps)) ** 2
        alpha = v / (1.0 - iou_val + v + eps)
        ciou = 1.0 - iou_val + rho2 / c2 + alpha * v
        out_ref[0] = jnp.nan_to_num(ciou, nan=0.0, posinf=10.0, neginf=0.0)

    f0_ref[0] = _focal(xt[4], t)
    f1_ref[0] = _focal(xt[9], t)
    no_ref[0] = (0.75 * _sigmoid(xt[4]) ** 2 * _softplus(xt[4])
                 + 0.75 * _sigmoid(xt[9]) ** 2 * _softplus(xt[9]))

    ch_iota = lax.broadcasted_iota(jnp.int32, (90, _RB, _S), 0)
    sel = ch_iota == (cid + 10)[None]
    contrib = _softplus(xt) - jnp.where(sel, xt, 0.0)
    cls_all = jnp.sum(contrib, axis=0)
    box10 = sum(_softplus(xt[k]) for k in range(10))
    cls_ref[0] = cls_all - box10


def _tc1(predictions, meta1, box6):
    grid = (_BS, _S // _RB)
    pf = jax.ShapeDtypeStruct((_BS, _S, _S), jnp.float32)
    blk = pl.BlockSpec((1, _RB, _S), lambda b, r: (b, r, 0))
    return pl.pallas_call(
        _tc1_body,
        grid=grid,
        in_specs=[
            pl.BlockSpec((1, _RB, _S, _Bb * 5 + _Cc), lambda b, r: (b, r, 0, 0)),
            blk,
            pl.BlockSpec((1, 6, _RB, _S), lambda b, r: (b, 0, r, 0)),
        ],
        out_specs=[blk] * 7,
        out_shape=[jax.ShapeDtypeStruct((_BS, _S, _S), jnp.int32),
                   pf, pf, pf, pf, pf, pf],
    )(predictions, meta1, box6)


_RB2 = 64


def _tc2_body(meta_ref, c0_ref, c1_ref, f0_ref, f1_ref, cls_ref, no_ref, out_ref):
    meta = meta_ref[0]
    obj = (meta & 1).astype(jnp.float32)
    resp = ((meta >> 1) & 1) == 1
    ciou = jnp.where(resp, c1_ref[0], c0_ref[0])
    fc = jnp.where(resp, f1_ref[0], f0_ref[0])
    partial = jnp.sum((_LC * ciou + fc + cls_ref[0]) * obj
                      + _LN * no_ref[0] * (1.0 - obj))

    @pl.when((pl.program_id(0) == 0) & (pl.program_id(1) == 0))
    def _():
        out_ref[0, 0] = 0.0
    out_ref[0, 0] += partial


def _tc2(meta2, c0, c1, f0, f1, clsm, nom):
    grid = (_BS, _S // _RB2)
    blk = pl.BlockSpec((1, _RB2, _S), lambda b, r: (b, r, 0))
    return pl.pallas_call(
        _tc2_body,
        grid=grid,
        in_specs=[blk] * 7,
        out_specs=pl.BlockSpec(memory_space=pltpu.SMEM,
                               block_shape=(1, 1),
                               index_map=lambda b, r: (0, 0)),
        out_shape=jax.ShapeDtypeStruct((1, 1), jnp.float32),
    )(meta2, c0, c1, f0, f1, clsm, nom)


def kernel(predictions, targets):
    tgt_pad = jnp.zeros((_BS, 512), jnp.float32)
    tgt_pad = tgt_pad.at[:, : _NT * 5].set(targets.reshape(_BS, _NT * 5))
    meta1, box6 = _sc1(tgt_pad)
    bjs, c0, c1, f0, f1, clsm, nom = _tc1(
        predictions,
        meta1.reshape(_BS, _S, _S),
        box6.reshape(_BS, 6, _S, _S),
    )
    meta2 = _sc2(tgt_pad, bjs.reshape(_BS * _S, _S))
    acc = _tc2(meta2.reshape(_BS, _S, _S), c0, c1, f0, f1, clsm, nom)
    return acc[0, 0] / _BS


# speedup vs baseline: 30.5652x; 1.0032x over previous
import functools
import math

import jax
import jax.numpy as jnp
from jax import lax
from jax.experimental import pallas as pl
from jax.experimental.pallas import tpu as pltpu
from jax.experimental.pallas import tpu_sc as plsc

_Bb = 2
_Cc = 80
_LC = 5.0
_LN = 0.1
_BS = 32
_S = 128
_NT = 100
_CELL = 1.0 / _S
_NCHUNK = 7
_NPAD = _NCHUNK * 16

_NC = 2
_NS = 16
_L = 16


def _sigmoid(x):
    return 1.0 / (1.0 + jnp.exp(-x))


def _mesh():
    return plsc.VectorSubcoreMesh(core_axis_name="c", subcore_axis_name="s",
                                  num_cores=_NC, num_subcores=_NS)


_SC_PARAMS = pltpu.CompilerParams(needs_layout_passes=False,
                                  use_tc_tiling_on_sc=False)


def _cells_of(tgt_v, c):
    lanes = lax.iota(jnp.int32, _L)
    lane = lanes + c * _L
    lv = jnp.minimum(lane, _NT - 1)
    base = lv * 5
    cx = plsc.load_gather(tgt_v, [base + 1])
    cy = plsc.load_gather(tgt_v, [base + 2])
    col_f = cx * float(_S)
    row_f = cy * float(_S)
    cols_c = jnp.clip(col_f.astype(jnp.int32), 0, _S - 1)
    rows_c = jnp.clip(row_f.astype(jnp.int32), 0, _S - 1)
    valid = lane < _NT
    return lane, lv, col_f, row_f, cols_c, rows_c, valid


def _sc1_body(tgt, meta_out, box_out, tgt_v, mbuf_v, meta_v, box_v):
    b = lax.axis_index("s") * _NC + lax.axis_index("c")
    zf = jnp.zeros((_L,), jnp.float32)
    zi = jnp.zeros((_L,), jnp.int32)

    def _zero_m(i, _):
        meta_v[pl.ds(i * _L, _L)] = zi
        return 0
    lax.fori_loop(0, (_S * _S) // _L, _zero_m, 0)

    def _zero_b(i, _):
        box_v[pl.ds(i * _L, _L)] = zf
        return 0
    lax.fori_loop(0, (4 * _S * _S) // _L, _zero_b, 0)

    pltpu.sync_copy(tgt.at[b], tgt_v)

    def _pc(c, _):
        lane, lv, col_f, row_f, cols_c, rows_c, valid = _cells_of(tgt_v, c)
        base = lv * 5
        cls_i = plsc.load_gather(tgt_v, [base + 0]).astype(jnp.int32)
        cx = plsc.load_gather(tgt_v, [base + 1])
        cy = plsc.load_gather(tgt_v, [base + 2])
        w = plsc.load_gather(tgt_v, [base + 3])
        h = plsc.load_gather(tgt_v, [base + 4])
        mval = 1 | (cls_i << 1)
        mbuf_v[pl.ds(c * _L, _L)] = mval
        cell = rows_c * _S + cols_c
        plsc.store_scatter(meta_v, [cell], mval, mask=valid)
        plsc.store_scatter(box_v, [cell], cx, mask=valid)
        plsc.store_scatter(box_v, [cell + _S * _S], cy, mask=valid)
        plsc.store_scatter(box_v, [cell + 2 * _S * _S], w, mask=valid)
        plsc.store_scatter(box_v, [cell + 3 * _S * _S], h, mask=valid)
        return 0
    lax.fori_loop(0, _NCHUNK, _pc, 0)

    def _pnx(c, _):
        _, _, col_f, _, cols_c, rows_c, valid = _cells_of(tgt_v, c)
        frac = col_f - cols_c.astype(jnp.float32)
        dx = jnp.where(frac >= 0.5, 1, -1).astype(jnp.int32)
        nc = cols_c + dx
        ok = valid & (nc >= 0) & (nc < _S)
        mval = mbuf_v[pl.ds(c * _L, _L)]
        plsc.store_scatter(meta_v, [rows_c * _S + jnp.clip(nc, 0, _S - 1)], mval, mask=ok)
        return 0
    lax.fori_loop(0, _NCHUNK, _pnx, 0)

    def _pny(c, _):
        _, _, _, row_f, cols_c, rows_c, valid = _cells_of(tgt_v, c)
        frac = row_f - rows_c.astype(jnp.float32)
        dy = jnp.where(frac >= 0.5, 1, -1).astype(jnp.int32)
        nr = rows_c + dy
        ok = valid & (nr >= 0) & (nr < _S)
        mval = mbuf_v[pl.ds(c * _L, _L)]
        plsc.store_scatter(meta_v, [jnp.clip(nr, 0, _S - 1) * _S + cols_c], mval, mask=ok)
        return 0
    lax.fori_loop(0, _NCHUNK, _pny, 0)

    pltpu.sync_copy(meta_v, meta_out.at[b])
    pltpu.sync_copy(box_v, box_out.at[b])


def _sc1(tgt_pad):
    return pl.kernel(
        _sc1_body,
        out_type=(
            jax.ShapeDtypeStruct((_BS, _S * _S), jnp.int32),
            jax.ShapeDtypeStruct((_BS, 4 * _S * _S), jnp.float32),
        ),
        mesh=_mesh(),
        compiler_params=_SC_PARAMS,
        scratch_types=[
            pltpu.VMEM((512,), jnp.float32),
            pltpu.VMEM((_NPAD,), jnp.int32),
            pltpu.VMEM((_S * _S,), jnp.int32),
            pltpu.VMEM((4 * _S * _S,), jnp.float32),
        ],
    )(tgt_pad)


def _sc2_body(tgt, bjs_rows, meta_out, tgt_v, idx_v, rows_v, mbuf_v, meta_v, sem):
    b = lax.axis_index("s") * _NC + lax.axis_index("c")
    zi = jnp.zeros((_L,), jnp.int32)

    def _zero_m(i, _):
        meta_v[pl.ds(i * _L, _L)] = zi
        return 0
    lax.fori_loop(0, (_S * _S) // _L, _zero_m, 0)

    pltpu.sync_copy(tgt.at[b], tgt_v)

    def _p1(c, _):
        _, _, _, _, cols_c, rows_c, _ = _cells_of(tgt_v, c)
        g = b * (_S * _S) + rows_c * _S + cols_c
        idx_v[0, pl.ds(c * _L, _L)] = g >> 7
        return 0
    lax.fori_loop(0, _NCHUNK, _p1, 0)

    pltpu.async_copy(bjs_rows.at[idx_v.at[0]], rows_v, sem).wait()

    def _p2(c, _):
        lane, _, _, _, cols_c, rows_c, valid = _cells_of(tgt_v, c)
        g = b * (_S * _S) + rows_c * _S + cols_c
        bjs = plsc.load_gather(rows_v, [lane, g & 127])
        mval = 1 | (bjs << 1)
        mbuf_v[pl.ds(c * _L, _L)] = mval
        plsc.store_scatter(meta_v, [rows_c * _S + cols_c], mval, mask=valid)
        return 0
    lax.fori_loop(0, _NCHUNK, _p2, 0)

    def _pnx(c, _):
        _, _, col_f, _, cols_c, rows_c, valid = _cells_of(tgt_v, c)
        frac = col_f - cols_c.astype(jnp.float32)
        dx = jnp.where(frac >= 0.5, 1, -1).astype(jnp.int32)
        nc = cols_c + dx
        ok = valid & (nc >= 0) & (nc < _S)
        mval = mbuf_v[pl.ds(c * _L, _L)]
        plsc.store_scatter(meta_v, [rows_c * _S + jnp.clip(nc, 0, _S - 1)], mval, mask=ok)
        return 0
    lax.fori_loop(0, _NCHUNK, _pnx, 0)

    def _pny(c, _):
        _, _, _, row_f, cols_c, rows_c, valid = _cells_of(tgt_v, c)
        frac = row_f - rows_c.astype(jnp.float32)
        dy = jnp.where(frac >= 0.5, 1, -1).astype(jnp.int32)
        nr = rows_c + dy
        ok = valid & (nr >= 0) & (nr < _S)
        mval = mbuf_v[pl.ds(c * _L, _L)]
        plsc.store_scatter(meta_v, [jnp.clip(nr, 0, _S - 1) * _S + cols_c], mval, mask=ok)
        return 0
    lax.fori_loop(0, _NCHUNK, _pny, 0)

    pltpu.sync_copy(meta_v, meta_out.at[b])


def _sc2(tgt_pad, bjs_rows):
    return pl.kernel(
        _sc2_body,
        out_type=jax.ShapeDtypeStruct((_BS, _S * _S), jnp.int32),
        mesh=_mesh(),
        compiler_params=_SC_PARAMS,
        scratch_types=[
            pltpu.VMEM((512,), jnp.float32),
            pltpu.VMEM((1, _NPAD), jnp.int32),
            pltpu.VMEM((_NPAD, 128), jnp.int32),
            pltpu.VMEM((_NPAD,), jnp.int32),
            pltpu.VMEM((_S * _S,), jnp.int32),
            pltpu.SemaphoreType.DMA,
        ],
    )(tgt_pad, bjs_rows)


_RB = 128


def _softplus(v):
    return jnp.maximum(v, 0.0) + jnp.log1p(jnp.exp(-jnp.abs(v)))


def _focal(logits, t):
    bce = (jnp.maximum(logits, 0.0) - logits * t
           + jnp.log1p(jnp.exp(-jnp.abs(logits))))
    p_t = t * _sigmoid(logits) + (1.0 - t) * _sigmoid(-logits)
    a_t = t * 0.25 + (1.0 - t) * 0.75
    return a_t * (1.0 - p_t) ** 2 * bce


def _tc1_body(pred_ref, meta_ref, box_ref, bjs_ref, a0_ref, a1_ref,
              cls_ref, no_ref):
    rb = pl.program_id(1)
    x = pred_ref[0]
    meta = meta_ref[0]
    box = box_ref[0]

    xt = jnp.transpose(x.reshape(_RB * _S, 90), (1, 0)).reshape(90, _RB, _S)

    cid = meta >> 1

    cols_g = lax.broadcasted_iota(jnp.int32, (_RB, _S), 1).astype(jnp.float32)
    rows_g = (lax.broadcasted_iota(jnp.int32, (_RB, _S), 0)
              + rb * _RB).astype(jnp.float32)

    gax, gay, gw0, gh0 = (box[k] for k in range(4))
    center = gw0 > 0.0
    grx = jnp.where(center, gax * float(_S) - cols_g, 0.0)
    gry = jnp.where(center, gay * float(_S) - rows_g, 0.0)
    gx1 = gax - gw0 * 0.5
    gy1 = gay - gh0 * 0.5
    gx2 = gax + gw0 * 0.5
    gy2 = gay + gh0 * 0.5
    garea = jnp.clip(gx2 - gx1, 0.0, None) * jnp.clip(gy2 - gy1, 0.0, None)

    pdec = []
    ious = []
    for j in range(2):
        p_cx = jnp.clip((_sigmoid(xt[5 * j + 0]) + cols_g) * _CELL, 0.0, 1.0)
        p_cy = jnp.clip((_sigmoid(xt[5 * j + 1]) + rows_g) * _CELL, 0.0, 1.0)
        p_w = jnp.clip(jnp.exp(jnp.clip(xt[5 * j + 2], -10.0, 10.0)) * _CELL, 1e-6, 1.0)
        p_h = jnp.clip(jnp.exp(jnp.clip(xt[5 * j + 3], -10.0, 10.0)) * _CELL, 1e-6, 1.0)
        pdec.append((p_cx, p_cy, p_w, p_h))
        px1 = p_cx - p_w * 0.5
        py1 = p_cy - p_h * 0.5
        px2 = p_cx + p_w * 0.5
        py2 = p_cy + p_h * 0.5
        ix1 = jnp.maximum(px1, gx1)
        iy1 = jnp.maximum(py1, gy1)
        ix2 = jnp.minimum(px2, gx2)
        iy2 = jnp.minimum(py2, gy2)
        inter = jnp.clip(ix2 - ix1, 0.0, None) * jnp.clip(iy2 - iy1, 0.0, None)
        union = (jnp.clip(px2 - px1, 0.0, None) * jnp.clip(py2 - py1, 0.0, None)
                 + garea - inter)
        ious.append(inter / (union + 1e-6))

    bjs_ref[0] = (ious[1] > ious[0]).astype(jnp.int32)

    gd_cx = (grx + cols_g) * _CELL
    gd_cy = (gry + rows_g) * _CELL

    eps = 1e-6
    gw = jnp.clip(gw0, eps, None)
    gh = jnp.clip(gh0, eps, None)
    ggx1 = gd_cx - gw * 0.5
    ggy1 = gd_cy - gh * 0.5
    ggx2 = gd_cx + gw * 0.5
    ggy2 = gd_cy + gh * 0.5
    atan_g = lax.atan2(gw, gh + eps)
    area_g = (ggx2 - ggx1) * (ggy2 - ggy1)

    t = jnp.where(center, jnp.maximum(ious[0], ious[1]), 0.0)
    for j, out_ref in ((0, a0_ref), (1, a1_ref)):
        pd_cx, pd_cy, pd_w, pd_h = pdec[j]
        pw = jnp.clip(pd_w, eps, None)
        ph = jnp.clip(pd_h, eps, None)
        px1 = pd_cx - pw * 0.5
        py1 = pd_cy - ph * 0.5
        px2 = pd_cx + pw * 0.5
        py2 = pd_cy + ph * 0.5
        ix1 = jnp.maximum(px1, ggx1)
        iy1 = jnp.maximum(py1, ggy1)
        ix2 = jnp.minimum(px2, ggx2)
        iy2 = jnp.minimum(py2, ggy2)
        inter = jnp.clip(ix2 - ix1, 0.0, None) * jnp.clip(iy2 - iy1, 0.0, None)
        union = (px2 - px1) * (py2 - py1) + area_g - inter
        iou_val = inter / (union + eps)
        rho2 = (pd_cx - gd_cx) ** 2 + (pd_cy - gd_cy) ** 2
        ex1 = jnp.minimum(px1, ggx1)
        ey1 = jnp.minimum(py1, ggy1)
        ex2 = jnp.maximum(px2, ggx2)
        ey2 = jnp.maximum(py2, ggy2)
        c2 = (ex2 - ex1) ** 2 + (ey2 - ey1) ** 2 + eps
        v = (4.0 / math.pi ** 2) * (atan_g - lax.atan2(pw, ph + eps)) ** 2
        alpha = v / (1.0 - iou_val + v + eps)
        ciou = 1.0 - iou_val + rho2 / c2 + alpha * v
        ciou = jnp.nan_to_num(ciou, nan=0.0, posinf=10.0, neginf=0.0)
        out_ref[0] = _LC * ciou + _focal(xt[4 + 5 * j], t)

    no_ref[0] = (0.75 * _sigmoid(xt[4]) ** 2 * _softplus(xt[4])
                 + 0.75 * _sigmoid(xt[9]) ** 2 * _softplus(xt[9]))

    ch_iota = lax.broadcasted_iota(jnp.int32, (90, _RB, _S), 0)
    sel = ch_iota == (cid + 10)[None]
    contrib = _softplus(xt) - jnp.where(sel, xt, 0.0)
    cls_all = jnp.sum(contrib, axis=0)
    box10 = sum(_softplus(xt[k]) for k in range(10))
    cls_ref[0] = cls_all - box10


def _tc1(predictions, meta1, box6):
    grid = (_BS, _S // _RB)
    pf = jax.ShapeDtypeStruct((_BS, _S, _S), jnp.float32)
    blk = pl.BlockSpec((1, _RB, _S), lambda b, r: (b, r, 0))
    return pl.pallas_call(
        _tc1_body,
        grid=grid,
        in_specs=[
            pl.BlockSpec((1, _RB, _S, _Bb * 5 + _Cc), lambda b, r: (b, r, 0, 0)),
            blk,
            pl.BlockSpec((1, 4, _RB, _S), lambda b, r: (b, 0, r, 0)),
        ],
        out_specs=[blk] * 5,
        out_shape=[jax.ShapeDtypeStruct((_BS, _S, _S), jnp.int32),
                   pf, pf, pf, pf],
    )(predictions, meta1, box6)


_RB2 = 64


def _tc2_body(meta_ref, a0_ref, a1_ref, cls_ref, no_ref, out_ref):
    meta = meta_ref[0]
    obj = (meta & 1).astype(jnp.float32)
    resp = ((meta >> 1) & 1) == 1
    a = jnp.where(resp, a1_ref[0], a0_ref[0])
    partial = jnp.sum((a + cls_ref[0]) * obj
                      + _LN * no_ref[0] * (1.0 - obj))

    @pl.when((pl.program_id(0) == 0) & (pl.program_id(1) == 0))
    def _():
        out_ref[0, 0] = 0.0
    out_ref[0, 0] += partial


def _tc2(meta2, a0, a1, clsm, nom):
    grid = (_BS, _S // _RB2)
    blk = pl.BlockSpec((1, _RB2, _S), lambda b, r: (b, r, 0))
    return pl.pallas_call(
        _tc2_body,
        grid=grid,
        in_specs=[blk] * 5,
        out_specs=pl.BlockSpec(memory_space=pltpu.SMEM,
                               block_shape=(1, 1),
                               index_map=lambda b, r: (0, 0)),
        out_shape=jax.ShapeDtypeStruct((1, 1), jnp.float32),
    )(meta2, a0, a1, clsm, nom)


def kernel(predictions, targets):
    tgt_pad = jnp.zeros((_BS, 512), jnp.float32)
    tgt_pad = tgt_pad.at[:, : _NT * 5].set(targets.reshape(_BS, _NT * 5))
    meta1, box4 = _sc1(tgt_pad)
    bjs, a0, a1, clsm, nom = _tc1(
        predictions,
        meta1.reshape(_BS, _S, _S),
        box4.reshape(_BS, 4, _S, _S),
    )
    meta2 = _sc2(tgt_pad, bjs.reshape(_BS * _S, _S))
    acc = _tc2(meta2.reshape(_BS, _S, _S), a0, a1, clsm, nom)
    return acc[0, 0] / _BS
